# baseline (speedup 1.0000x reference)
import functools

import jax
import jax.numpy as jnp
from jax import lax
from jax.experimental import pallas as pl
from jax.experimental.pallas import tpu as pltpu
from jax.experimental.pallas import tpu_sc as plsc

_EMB = 32
_NSH = 4
_NW = 32
_L = 16
_C2 = 256
_G = 64


def _build(b_total, nlines):
    c = b_total // _NW
    nsub = c // _C2
    ngrp = _C2 // _L
    nk = _C2 // _G

    mesh = plsc.VectorSubcoreMesh(core_axis_name="c", subcore_axis_name="s")

    @functools.partial(
        pl.kernel,
        mesh=mesh,
        out_type=jax.ShapeDtypeStruct((b_total * _EMB,), jnp.float32),
        scratch_types=[
            pltpu.VMEM((c,), jnp.int32),
            pltpu.VMEM((_C2,), jnp.int32),
            pltpu.VMEM((_C2,), jnp.int32),
            pltpu.VMEM((_C2, 128), jnp.float32),
            pltpu.VMEM((_C2, 128), jnp.float32),
            pltpu.VMEM((_C2 * _EMB,), jnp.float32),
            pltpu.SemaphoreType.DMA,
            pltpu.SemaphoreType.DMA,
        ],
        compiler_params=pltpu.CompilerParams(needs_layout_passes=False),
    )
    def lookup(ids_h, tbl, out_h,
               ids_v, ll0, ll1, gbuf0, gbuf1, obuf, gs0, gs1):
        llA = (ll0, ll1)
        gbufA = (gbuf0, gbuf1)
        gsA = (gs0, gs1)

        wid = lax.axis_index("s") * 2 + lax.axis_index("c")
        wbase = wid * c
        pltpu.sync_copy(ids_h.at[pl.ds(wbase, c)], ids_v)

        def step(s, par):
            sbase = jnp.minimum(s, nsub - 1) * _C2

            def lgroup(g, _):
                v = ids_v[pl.ds(sbase + g * _L, _L)]
                line = (v & (_NSH - 1)) * nlines + lax.shift_right_logical(v, 4)
                llA[par][pl.ds(g * _L, _L)] = line
                return 0

            lax.fori_loop(0, ngrp, lgroup, 0)

            @pl.when(s < nsub)
            def _():
                for k in range(nk):
                    pltpu.make_async_copy(
                        tbl.at[llA[par].at[pl.ds(k * _G, _G)]],
                        gbufA[par].at[pl.ds(k * _G, _G)],
                        gsA[par],
                    ).start()

            @pl.when(s >= 1)
            def _():
                for k in range(nk):
                    pltpu.make_async_copy(
                        tbl.at[llA[1 - par].at[pl.ds(k * _G, _G)]],
                        gbufA[1 - par].at[pl.ds(k * _G, _G)],
                        gsA[1 - par],
                    ).wait()

            spbase = jnp.maximum(s - 1, 0) * _C2

            def egroup(g, _):
                v = ids_v[pl.ds(spbase + g * _L, _L)]
                col0 = (lax.shift_right_logical(v, 2) & (_NSH - 1)) * _EMB
                for l in range(_L):
                    cb = col0[l]
                    e = g * _L + l
                    obuf[pl.ds(e * _EMB, _L)] = \
                        gbufA[1 - par][e, pl.ds(cb, _L)]
                    obuf[pl.ds(e * _EMB + _L, _L)] = \
                        gbufA[1 - par][e, pl.ds(cb + _L, _L)]
                return 0

            lax.fori_loop(0, ngrp, egroup, 0)

            @pl.when(s >= 1)
            def _():
                pltpu.sync_copy(
                    obuf,
                    out_h.at[pl.ds((wbase + spbase) * _EMB, _C2 * _EMB)])

        def dbody(i, carry):
            step(2 * i, 0)
            step(2 * i + 1, 1)
            return carry

        lax.fori_loop(0, (nsub + 2) // 2, dbody, 0)

    return lookup


def kernel(inputs, emb_0, emb_1, emb_2, emb_3):
    batch, steps = inputs.shape
    b_total = batch * steps
    ids = inputs.reshape(b_total)
    nlines = emb_0.shape[0] // _NSH
    tbl = jnp.concatenate(
        (emb_0, emb_1, emb_2, emb_3), axis=0).reshape(_NSH * nlines, _NSH * _EMB)
    out = _build(b_total, nlines)(ids, tbl)
    return out.reshape(batch, steps, _EMB)


# speedup vs baseline: 1.2024x; 1.2024x over previous
import functools

import jax
import jax.numpy as jnp
from jax import lax
from jax.experimental import pallas as pl
from jax.experimental.pallas import tpu as pltpu
from jax.experimental.pallas import tpu_sc as plsc

_EMB = 32
_NSH = 4
_NW = 32
_L = 16
_C2 = 256
_G = 16
_LCAP = _C2 + _NSH * _G
_GLEN = _C2 + _G


def _build(b_total):
    c = b_total // _NW
    nsub = c // _C2
    ngrp = _C2 // _L
    nk = _C2 // _G

    mesh = plsc.VectorSubcoreMesh(core_axis_name="c", subcore_axis_name="s")

    @functools.partial(
        pl.kernel,
        mesh=mesh,
        out_type=jax.ShapeDtypeStruct((b_total * _EMB,), jnp.float32),
        scratch_types=[
            pltpu.VMEM((c,), jnp.int32),
            pltpu.VMEM((_GLEN,), jnp.int32),
            pltpu.VMEM((_GLEN,), jnp.int32),
            pltpu.VMEM((_GLEN,), jnp.int32),
            pltpu.VMEM((_GLEN,), jnp.int32),
            pltpu.VMEM((_GLEN,), jnp.int32),
            pltpu.VMEM((_GLEN,), jnp.int32),
            pltpu.VMEM((_GLEN,), jnp.int32),
            pltpu.VMEM((_GLEN,), jnp.int32),
            pltpu.VMEM((_C2,), jnp.int32),
            pltpu.VMEM((_C2,), jnp.int32),
            pltpu.VMEM((_L,), jnp.int32),
            pltpu.VMEM((_L,), jnp.int32),
            pltpu.VMEM((_LCAP, 128), jnp.float32),
            pltpu.VMEM((_LCAP, 128), jnp.float32),
            pltpu.VMEM((_C2 * _EMB,), jnp.float32),
            pltpu.SemaphoreType.DMA,
            pltpu.SemaphoreType.DMA,
        ],
        compiler_params=pltpu.CompilerParams(needs_layout_passes=False),
    )
    def lookup(ids_h, t0, t1, t2, t3, out_h,
               ids_v, gl00, gl01, gl02, gl03, gl10, gl11, gl12, gl13,
               rel0, rel1, stv0, stv1, gbuf0, gbuf1, obuf, gs0, gs1):
        tbls = (t0, t1, t2, t3)
        glsA = ((gl00, gl01, gl02, gl03), (gl10, gl11, gl12, gl13))
        relA = (rel0, rel1)
        stvA = (stv0, stv1)
        gbufA = (gbuf0, gbuf1)
        gsA = (gs0, gs1)

        wid = lax.axis_index("s") * 2 + lax.axis_index("c")
        wbase = wid * c
        pltpu.sync_copy(ids_h.at[pl.ds(wbase, c)], ids_v)

        iota = lax.broadcasted_iota(jnp.int32, (_L,), 0)
        full = iota >= 0
        zero16 = jnp.zeros((_L,), jnp.int32)

        def compact(sbase, gls, rel_v, st_v):
            def cgroup(g, offs):
                v = ids_v[pl.ds(sbase + g * _L, _L)]
                sh = v & (_NSH - 1)
                line = lax.shift_right_logical(v, 4)
                rel = zero16
                new = []
                for t in range(_NSH):
                    m = sh == t
                    cs = plsc.cumsum(m.astype(jnp.int32))
                    pos = offs[t] + cs - 1
                    plsc.store_scatter(gls[t], [pos], line, mask=m)
                    rel = jnp.where(m, pos, rel)
                    new.append(offs[t] + cs[_L - 1])
                rel_v[pl.ds(g * _L, _L)] = rel
                return tuple(new)

            offs = lax.fori_loop(0, ngrp, cgroup, (jnp.int32(0),) * _NSH)
            for t in range(_NSH):
                plsc.store_scatter(gls[t], [offs[t] + iota], zero16, mask=full)
            starts = []
            acc = jnp.int32(0)
            for t in range(_NSH):
                starts.append(acc)
                acc = acc + ((offs[t] + (_G - 1)) & ~(_G - 1))
            sv = zero16
            for t in range(_NSH):
                sv = jnp.where(iota == t, starts[t], sv)
            st_v[pl.ds(0, _L)] = sv
            return offs, starts

        def dma_each(offs, starts, gls, gbuf, gsem, op):
            for t in range(_NSH):
                def body(k, _, t=t, n=offs[t], st=starts[t]):
                    @pl.when(k * _G < n)
                    def _():
                        cp = pltpu.make_async_copy(
                            tbls[t].at[gls[t].at[pl.ds(k * _G, _G)]],
                            gbuf.at[pl.ds(st + k * _G, _G)],
                            gsem,
                        )
                        cp.start() if op == "start" else cp.wait()
                    return 0
                lax.fori_loop(0, nk, body, 0)

        def extract(sbase, rel_v, st_v, gbuf):
            def egroup(g, _):
                v = ids_v[pl.ds(sbase + g * _L, _L)]
                sh = v & (_NSH - 1)
                sub = lax.shift_right_logical(v, 2) & (_NSH - 1)
                rel = rel_v[pl.ds(g * _L, _L)]
                slot = plsc.load_gather(st_v, [sh]) + rel
                slot = jnp.minimum(jnp.maximum(slot, 0), _LCAP - 1)
                col0 = sub * _EMB
                for l in range(_L):
                    r = slot[l]
                    cb = col0[l]
                    e = (g * _L + l) * _EMB
                    obuf[pl.ds(e, _L)] = gbuf[r, pl.ds(cb, _L)]
                    obuf[pl.ds(e + _L, _L)] = gbuf[r, pl.ds(cb + _L, _L)]
                return 0

            lax.fori_loop(0, ngrp, egroup, 0)

        def step(s, carry, par):
            n_prev = carry[:_NSH]
            st_prev = carry[_NSH:]
            s_eff = jnp.minimum(s, nsub - 1)
            offs, starts = compact(s_eff * _C2, glsA[par], relA[par], stvA[par])
            n_live = tuple(jnp.where(s < nsub, offs[t], 0) for t in range(_NSH))
            dma_each(n_live, starts, glsA[par], gbufA[par], gsA[par], "start")
            dma_each(n_prev, st_prev, glsA[1 - par], gbufA[1 - par],
                     gsA[1 - par], "wait")
            sp = jnp.maximum(s - 1, 0)
            extract(sp * _C2, relA[1 - par], stvA[1 - par], gbufA[1 - par])

            @pl.when(s >= 1)
            def _():
                pltpu.sync_copy(
                    obuf,
                    out_h.at[pl.ds((wbase + sp * _C2) * _EMB, _C2 * _EMB)])
            return n_live + tuple(starts)

        def dbody(i, carry):
            carry = step(2 * i, carry, 0)
            carry = step(2 * i + 1, carry, 1)
            return carry

        lax.fori_loop(0, (nsub + 2) // 2, dbody, (jnp.int32(0),) * (2 * _NSH))

    return lookup


def kernel(inputs, emb_0, emb_1, emb_2, emb_3):
    batch, steps = inputs.shape
    b_total = batch * steps
    ids = inputs.reshape(b_total)
    lines = emb_0.shape[0] // _NSH
    tbls = [e.reshape(lines, _NSH * _EMB) for e in (emb_0, emb_1, emb_2, emb_3)]
    out = _build(b_total)(ids, *tbls)
    return out.reshape(batch, steps, _EMB)
